# baseline (speedup 1.0000x reference)
import jax
import jax.numpy as jnp
from jax.experimental import pallas as pl
from jax.experimental.pallas import tpu as pltpu

_B, _C, _DHW, _S = 8, 16, 64, 8
_POOL = _DHW // _S
_SLAB = _S ** 3
_HALF = _B * _SLAB
_N = 2 * _HALF
_INV_T = 10.0
_SC2 = 14.426950408889634
_EPS = 1e-12

_DBLK = 16
_ND = _DBLK // _POOL
_R = _SLAB
_NB = _HALF // _R
_NCHUNK = 4


def _pool_body(x1_ref, x2_ref, o_ref):
    wi = jax.lax.broadcasted_iota(jnp.int32, (_DHW, _S), 0)
    wo = jax.lax.broadcasted_iota(jnp.int32, (_DHW, _S), 1)
    pm = jnp.where(wi // _POOL == wo, 1.0 / (_POOL ** 3), 0.0).astype(jnp.float32)

    for v, x_ref in ((0, x1_ref), (1, x2_ref)):
        x = x_ref[0]
        hs = []
        for k in range(_ND):
            acc = x[:, _POOL * k]
            for d in range(1, _POOL):
                acc = acc + x[:, _POOL * k + d]
            hs.append(acc.reshape(_C, _S, _POOL, _DHW).sum(axis=2))
        xh = jnp.stack(hs, axis=1)
        y = jnp.dot(xh.reshape(_C * _ND * _S, _DHW), pm,
                    preferred_element_type=jnp.float32,
                    precision=jax.lax.Precision.HIGHEST)
        o_ref[v, 0] = y.reshape(_C, _ND, _S, _S)


def _loss_body(pf_ref, w1_ref, b1_ref, w2_ref, b2_ref, o_ref,
               fmat_ref, row_ref, col_ref):
    i = pl.program_id(0)
    j = pl.program_id(1)
    s = i * _NB + j

    def _mlp(x):
        h = jnp.dot(w1_ref[...], x, preferred_element_type=jnp.float32,
                    precision=jax.lax.Precision.HIGHEST) + b1_ref[...]
        h = jnp.maximum(h, 0.0)
        f = jnp.dot(w2_ref[...], h, preferred_element_type=jnp.float32,
                    precision=jax.lax.Precision.HIGHEST) + b2_ref[...]
        nrm = jnp.sqrt(jnp.sum(f * f, axis=0, keepdims=True))
        return f / jnp.maximum(nrm, _EPS)

    @pl.when(s == 0)
    def _():
        for v in range(2):
            for b in range(_B):
                t = v * _B + b
                fmat_ref[:, t * _SLAB:(t + 1) * _SLAB] = (
                    _mlp(pf_ref[v, b]).astype(jnp.bfloat16))
        col_ref[...] = jnp.zeros_like(col_ref)

    fr = _mlp(pf_ref[i, j])
    fp = _mlp(pf_ref[1 - i, j])
    frb = (fr * _SC2).astype(jnp.bfloat16)

    row_ref[...] = jnp.zeros_like(row_ref)
    for t in range(2 * _NB):
        @pl.when(s <= t)
        def _():
            fbt = fmat_ref[:, t * _SLAB:(t + 1) * _SLAB]
            lg = jax.lax.dot_general(frb, fbt, (((0,), (0,)), ((), ())),
                                     preferred_element_type=jnp.float32)
            e = jnp.exp2(lg)
            row_ref[...] = row_ref[...] + jnp.sum(e, axis=1, keepdims=True)
            if t > 0:
                @pl.when(s < t)
                def _():
                    col_ref[:, t * _SLAB:(t + 1) * _SLAB] = (
                        col_ref[:, t * _SLAB:(t + 1) * _SLAB]
                        + jnp.sum(e, axis=0, keepdims=True))

    cslice = col_ref[:, pl.ds(pl.multiple_of(s * _SLAB, _SLAB), _SLAB)]
    ones11 = jnp.ones((1, 1), dtype=jnp.float32)
    ct = jax.lax.dot_general(cslice, ones11, (((0,), (0,)), ((), ())),
                             preferred_element_type=jnp.float32,
                             precision=jax.lax.Precision.HIGHEST)
    e_sum = row_ref[...] + ct

    a = frb.astype(jnp.float32)
    bt2 = fr.astype(jnp.bfloat16).astype(jnp.float32)
    ones_c = jnp.ones((_C, 1), dtype=jnp.float32)
    dg = jax.lax.dot_general(a * bt2, ones_c, (((0,), (0,)), ((), ())),
                             preferred_element_type=jnp.float32,
                             precision=jax.lax.Precision.HIGHEST)
    s_off = e_sum - jnp.exp2(dg)

    partial = jnp.sum(jnp.log(s_off)) - _INV_T * jnp.sum(fr * fp)

    @pl.when(s == 0)
    def _():
        o_ref[0, 0, 0] = partial

    @pl.when(s > 0)
    def _():
        o_ref[0, 0, 0] = o_ref[0, 0, 0] + partial


def kernel(p1, p2, w1, b1, w2, b2):
    pooled = pl.pallas_call(
        _pool_body,
        grid=(_B, _DHW // _DBLK),
        in_specs=[
            pl.BlockSpec((1, _C, _DBLK, _DHW, _DHW), lambda b, d: (b, 0, d, 0, 0)),
            pl.BlockSpec((1, _C, _DBLK, _DHW, _DHW), lambda b, d: (b, 0, d, 0, 0)),
        ],
        out_specs=pl.BlockSpec((2, 1, _C, _ND, _S, _S),
                               lambda b, d: (0, b, 0, d, 0, 0)),
        out_shape=jax.ShapeDtypeStruct((2, _B, _C, _S, _S, _S), jnp.float32),
        compiler_params=pltpu.CompilerParams(
            dimension_semantics=("arbitrary", "arbitrary"),
        ),
        name="region_pool",
    )(p1, p2)

    pooled = pooled.reshape(2, _B, _C, _SLAB)

    partials = pl.pallas_call(
        _loss_body,
        grid=(2, _NB),
        in_specs=[
            pl.BlockSpec((2, _B, _C, _SLAB), lambda i, j: (0, 0, 0, 0)),
            pl.BlockSpec((_C, _C), lambda i, j: (0, 0)),
            pl.BlockSpec((_C, 1), lambda i, j: (0, 0)),
            pl.BlockSpec((_C, _C), lambda i, j: (0, 0)),
            pl.BlockSpec((_C, 1), lambda i, j: (0, 0)),
        ],
        out_specs=pl.BlockSpec((1, 1, 1), lambda i, j: (0, 0, 0),
                               memory_space=pltpu.SMEM),
        out_shape=jax.ShapeDtypeStruct((1, 1, 1), jnp.float32),
        scratch_shapes=[pltpu.VMEM((_C, _N), jnp.bfloat16),
                        pltpu.VMEM((_R, 1), jnp.float32),
                        pltpu.VMEM((1, _N), jnp.float32)],
        compiler_params=pltpu.CompilerParams(
            dimension_semantics=("arbitrary", "arbitrary"),
            vmem_limit_bytes=48 * 1024 * 1024,
        ),
        name="head_infonce_loss",
    )(pooled, w1, b1.reshape(_C, 1), w2, b2.reshape(_C, 1))

    return partials[0, 0, 0] / _N


# speedup vs baseline: 1.1594x; 1.1594x over previous
import jax
import jax.numpy as jnp
from jax.experimental import pallas as pl
from jax.experimental.pallas import tpu as pltpu

_B, _C, _DHW, _S = 8, 16, 64, 8
_POOL = _DHW // _S
_SLAB = _S ** 3
_HALF = _B * _SLAB
_N = 2 * _HALF
_INV_T = 10.0
_SC2 = 14.426950408889634
_EPS = 1e-12

_DBLK = 16
_ND = _DBLK // _POOL
_R = _SLAB
_NB = _HALF // _R
_NCHUNK = 4


def _pool_body(x1_ref, x2_ref, o_ref):
    wi = jax.lax.broadcasted_iota(jnp.int32, (_DHW, _S), 0)
    wo = jax.lax.broadcasted_iota(jnp.int32, (_DHW, _S), 1)
    pm = jnp.where(wi // _POOL == wo, 1.0 / (_POOL ** 3), 0.0).astype(jnp.float32)

    for v, x_ref in ((0, x1_ref), (1, x2_ref)):
        x = x_ref[0]
        hs = []
        for k in range(_ND):
            acc = x[:, _POOL * k]
            for d in range(1, _POOL):
                acc = acc + x[:, _POOL * k + d]
            hs.append(acc.reshape(_C, _S, _POOL, _DHW).sum(axis=2))
        xh = jnp.stack(hs, axis=1)
        y = jnp.dot(xh.reshape(_C * _ND * _S, _DHW), pm,
                    preferred_element_type=jnp.float32,
                    precision=jax.lax.Precision.HIGHEST)
        o_ref[v, 0] = y.reshape(_C, _ND, _S, _S)


def _loss_body(pf_ref, w1_ref, b1_ref, w2_ref, b2_ref, o_ref, fmat_ref):
    i = pl.program_id(0)
    j = pl.program_id(1)
    s = i * _NB + j

    def _mlp(x):
        h = jnp.dot(w1_ref[...], x, preferred_element_type=jnp.float32,
                    precision=jax.lax.Precision.HIGHEST) + b1_ref[...]
        h = jnp.maximum(h, 0.0)
        f = jnp.dot(w2_ref[...], h, preferred_element_type=jnp.float32,
                    precision=jax.lax.Precision.HIGHEST) + b2_ref[...]
        nrm = jnp.sqrt(jnp.sum(f * f, axis=0, keepdims=True))
        return f / jnp.maximum(nrm, _EPS)

    @pl.when(s == 0)
    def _():
        for v in range(2):
            for b in range(_B):
                t = v * _B + b
                fmat_ref[:, t * _SLAB:(t + 1) * _SLAB] = (
                    _mlp(pf_ref[v, b]).astype(jnp.bfloat16))

    fr = _mlp(pf_ref[i, j])
    fp = _mlp(pf_ref[1 - i, j])
    frb = (fr * _SC2).astype(jnp.bfloat16)

    cw = _N // _NCHUNK
    e_sum = jnp.zeros((_R, 1), dtype=jnp.float32)
    for q in range(_NCHUNK):
        fbq = fmat_ref[:, q * cw:(q + 1) * cw]
        lg = jax.lax.dot_general(frb, fbq, (((0,), (0,)), ((), ())),
                                 preferred_element_type=jnp.float32)
        e = jnp.exp2(lg.astype(jnp.bfloat16))
        e_sum = e_sum + jnp.sum(e, axis=1, keepdims=True,
                                dtype=jnp.float32)

    a = frb.astype(jnp.float32)
    bt2 = fr.astype(jnp.bfloat16).astype(jnp.float32)
    ones_c = jnp.ones((_C, 1), dtype=jnp.float32)
    dg = jax.lax.dot_general(a * bt2, ones_c, (((0,), (0,)), ((), ())),
                             preferred_element_type=jnp.float32,
                             precision=jax.lax.Precision.HIGHEST)
    dgb = dg.astype(jnp.bfloat16).astype(jnp.float32)
    s_off = e_sum - jnp.exp2(dgb)

    partial = jnp.sum(jnp.log(s_off)) - _INV_T * jnp.sum(fr * fp)

    @pl.when(s == 0)
    def _():
        o_ref[0, 0, 0] = partial

    @pl.when(s > 0)
    def _():
        o_ref[0, 0, 0] = o_ref[0, 0, 0] + partial


def kernel(p1, p2, w1, b1, w2, b2):
    pooled = pl.pallas_call(
        _pool_body,
        grid=(_B, _DHW // _DBLK),
        in_specs=[
            pl.BlockSpec((1, _C, _DBLK, _DHW, _DHW), lambda b, d: (b, 0, d, 0, 0)),
            pl.BlockSpec((1, _C, _DBLK, _DHW, _DHW), lambda b, d: (b, 0, d, 0, 0)),
        ],
        out_specs=pl.BlockSpec((2, 1, _C, _ND, _S, _S),
                               lambda b, d: (0, b, 0, d, 0, 0)),
        out_shape=jax.ShapeDtypeStruct((2, _B, _C, _S, _S, _S), jnp.float32),
        compiler_params=pltpu.CompilerParams(
            dimension_semantics=("arbitrary", "arbitrary"),
        ),
        name="region_pool",
    )(p1, p2)

    pooled = pooled.reshape(2, _B, _C, _SLAB)

    partials = pl.pallas_call(
        _loss_body,
        grid=(2, _NB),
        in_specs=[
            pl.BlockSpec((2, _B, _C, _SLAB), lambda i, j: (0, 0, 0, 0)),
            pl.BlockSpec((_C, _C), lambda i, j: (0, 0)),
            pl.BlockSpec((_C, 1), lambda i, j: (0, 0)),
            pl.BlockSpec((_C, _C), lambda i, j: (0, 0)),
            pl.BlockSpec((_C, 1), lambda i, j: (0, 0)),
        ],
        out_specs=pl.BlockSpec((1, 1, 1), lambda i, j: (0, 0, 0),
                               memory_space=pltpu.SMEM),
        out_shape=jax.ShapeDtypeStruct((1, 1, 1), jnp.float32),
        scratch_shapes=[pltpu.VMEM((_C, _N), jnp.bfloat16)],
        compiler_params=pltpu.CompilerParams(
            dimension_semantics=("arbitrary", "arbitrary"),
            vmem_limit_bytes=48 * 1024 * 1024,
        ),
        name="head_infonce_loss",
    )(pooled, w1, b1.reshape(_C, 1), w2, b2.reshape(_C, 1))

    return partials[0, 0, 0] / _N


# speedup vs baseline: 1.2126x; 1.0459x over previous
import jax
import jax.numpy as jnp
from jax.experimental import pallas as pl
from jax.experimental.pallas import tpu as pltpu

_B, _C, _DHW, _S = 8, 16, 64, 8
_POOL = _DHW // _S
_SLAB = _S ** 3
_HALF = _B * _SLAB
_N = 2 * _HALF
_INV_T = 10.0
_SC2 = 14.426950408889634
_EPS = 1e-12

_DBLK = 16
_ND = _DBLK // _POOL
_R = _SLAB
_NB = _HALF // _R
_NCHUNK = 8


def _pool_body(x1_ref, x2_ref, o_ref):
    wi = jax.lax.broadcasted_iota(jnp.int32, (_DHW, _S), 0)
    wo = jax.lax.broadcasted_iota(jnp.int32, (_DHW, _S), 1)
    pm = jnp.where(wi // _POOL == wo, 1.0 / (_POOL ** 3), 0.0).astype(jnp.float32)

    for v, x_ref in ((0, x1_ref), (1, x2_ref)):
        hs = []
        for k in range(_ND):
            acc = x_ref[0, :, _POOL * k]
            for d in range(1, _POOL):
                acc = acc + x_ref[0, :, _POOL * k + d]
            hs.append(acc.reshape(_C, _S, _POOL, _DHW).sum(axis=2))
        xh = jnp.stack(hs, axis=1)
        y = jnp.dot(xh.reshape(_C * _ND * _S, _DHW), pm,
                    preferred_element_type=jnp.float32,
                    precision=jax.lax.Precision.HIGHEST)
        o_ref[v, 0] = y.reshape(_C, _ND, _S, _S)


def _loss_body(pf_ref, w1_ref, b1_ref, w2_ref, b2_ref, o_ref, fmat_ref):
    i = pl.program_id(0)
    j = pl.program_id(1)
    s = i * _NB + j

    def _mlp(x):
        h = jnp.dot(w1_ref[...], x,
                    preferred_element_type=jnp.float32) + b1_ref[...]
        h = jnp.maximum(h, 0.0)
        f = jnp.dot(w2_ref[...], h,
                    preferred_element_type=jnp.float32) + b2_ref[...]
        nrm = jnp.sqrt(jnp.sum(f * f, axis=0, keepdims=True))
        return f / jnp.maximum(nrm, _EPS)

    @pl.when(s == 0)
    def _():
        xcat = jnp.concatenate(
            [pf_ref[v, b] for v in range(2) for b in range(_B)], axis=1)
        fmat_ref[...] = _mlp(xcat).astype(jnp.bfloat16)

    xp = jnp.concatenate([pf_ref[i, j], pf_ref[1 - i, j]], axis=1)
    f2 = _mlp(xp)
    fr = f2[:, :_SLAB]
    fp = f2[:, _SLAB:]
    frb = (fr * _SC2).astype(jnp.bfloat16)

    cw = _N // _NCHUNK
    e_sum = jnp.zeros((_R, 1), dtype=jnp.float32)
    for q in range(_NCHUNK):
        fbq = fmat_ref[:, q * cw:(q + 1) * cw]
        lg = jax.lax.dot_general(frb, fbq, (((0,), (0,)), ((), ())),
                                 preferred_element_type=jnp.float32)
        e_sum = e_sum + jnp.sum(jnp.exp2(lg), axis=1, keepdims=True)

    a = frb.astype(jnp.float32)
    bt2 = fr.astype(jnp.bfloat16).astype(jnp.float32)
    ones_c = jnp.ones((_C, 1), dtype=jnp.float32)
    dg = jax.lax.dot_general(a * bt2, ones_c, (((0,), (0,)), ((), ())),
                             preferred_element_type=jnp.float32)
    s_off = e_sum - jnp.exp2(dg)

    partial = jnp.sum(jnp.log(s_off)) - _INV_T * jnp.sum(fr * fp)

    @pl.when(s == 0)
    def _():
        o_ref[0, 0, 0] = partial

    @pl.when(s > 0)
    def _():
        o_ref[0, 0, 0] = o_ref[0, 0, 0] + partial


def kernel(p1, p2, w1, b1, w2, b2):
    pooled = pl.pallas_call(
        _pool_body,
        grid=(_B, _DHW // _DBLK),
        in_specs=[
            pl.BlockSpec((1, _C, _DBLK, _DHW, _DHW), lambda b, d: (b, 0, d, 0, 0)),
            pl.BlockSpec((1, _C, _DBLK, _DHW, _DHW), lambda b, d: (b, 0, d, 0, 0)),
        ],
        out_specs=pl.BlockSpec((2, 1, _C, _ND, _S, _S),
                               lambda b, d: (0, b, 0, d, 0, 0)),
        out_shape=jax.ShapeDtypeStruct((2, _B, _C, _S, _S, _S), jnp.float32),
        compiler_params=pltpu.CompilerParams(
            dimension_semantics=("arbitrary", "arbitrary"),
            vmem_limit_bytes=56 * 1024 * 1024,
        ),
        name="region_pool",
    )(p1, p2)

    pooled = pooled.reshape(2, _B, _C, _SLAB)

    partials = pl.pallas_call(
        _loss_body,
        grid=(2, _NB),
        in_specs=[
            pl.BlockSpec((2, _B, _C, _SLAB), lambda i, j: (0, 0, 0, 0)),
            pl.BlockSpec((_C, _C), lambda i, j: (0, 0)),
            pl.BlockSpec((_C, 1), lambda i, j: (0, 0)),
            pl.BlockSpec((_C, _C), lambda i, j: (0, 0)),
            pl.BlockSpec((_C, 1), lambda i, j: (0, 0)),
        ],
        out_specs=pl.BlockSpec((1, 1, 1), lambda i, j: (0, 0, 0),
                               memory_space=pltpu.SMEM),
        out_shape=jax.ShapeDtypeStruct((1, 1, 1), jnp.float32),
        scratch_shapes=[pltpu.VMEM((_C, _N), jnp.bfloat16)],
        compiler_params=pltpu.CompilerParams(
            dimension_semantics=("arbitrary", "arbitrary"),
            vmem_limit_bytes=48 * 1024 * 1024,
        ),
        name="head_infonce_loss",
    )(pooled, w1, b1.reshape(_C, 1), w2, b2.reshape(_C, 1))

    return partials[0, 0, 0] / _N


# speedup vs baseline: 1.2233x; 1.0088x over previous
import jax
import jax.numpy as jnp
from jax.experimental import pallas as pl
from jax.experimental.pallas import tpu as pltpu

_B, _C, _DHW, _S = 8, 16, 64, 8
_POOL = _DHW // _S
_SLAB = _S ** 3
_HALF = _B * _SLAB
_N = 2 * _HALF
_INV_T = 10.0
_SC2 = 14.426950408889634
_EPS = 1e-12

_DBLK = 16
_ND = _DBLK // _POOL
_R = _SLAB
_NB = _HALF // _R
_NCHUNK = 8


def _pool_body(x1_ref, x2_ref, o_ref):
    wi = jax.lax.broadcasted_iota(jnp.int32, (_DHW, _S), 0)
    wo = jax.lax.broadcasted_iota(jnp.int32, (_DHW, _S), 1)
    pm = jnp.where(wi // _POOL == wo, 1.0 / (_POOL ** 3), 0.0).astype(jnp.float32)

    for v, x_ref in ((0, x1_ref), (1, x2_ref)):
        hs = []
        for k in range(_ND):
            acc = x_ref[0, :, _POOL * k]
            for d in range(1, _POOL):
                acc = acc + x_ref[0, :, _POOL * k + d]
            hs.append(acc.reshape(_C, _S, _POOL, _DHW).sum(axis=2))
        xh = jnp.stack(hs, axis=1)
        y = jnp.dot(xh.reshape(_C * _ND * _S, _DHW), pm,
                    preferred_element_type=jnp.float32,
                    precision=jax.lax.Precision.HIGHEST)
        o_ref[v, 0] = y.reshape(_C, _ND, _S, _S)


def _loss_body(pf_ref, w1_ref, b1_ref, w2_ref, b2_ref, o_ref, fmat_ref):
    i = pl.program_id(0)
    j = pl.program_id(1)
    s = i * _NB + j

    def _mlp(x):
        h = jnp.dot(w1_ref[...], x,
                    preferred_element_type=jnp.float32) + b1_ref[...]
        h = jnp.maximum(h, 0.0)
        f = jnp.dot(w2_ref[...], h,
                    preferred_element_type=jnp.float32) + b2_ref[...]
        nrm = jnp.sqrt(jnp.sum(f * f, axis=0, keepdims=True))
        return f / jnp.maximum(nrm, _EPS)

    @pl.when(s == 0)
    def _():
        xcat = jnp.concatenate(
            [pf_ref[v, b] for v in range(2) for b in range(_B)], axis=1)
        fmat_ref[...] = _mlp(xcat).astype(jnp.bfloat16)

    xp = jnp.concatenate([pf_ref[i, j], pf_ref[1 - i, j]], axis=1)
    f2 = _mlp(xp)
    fr = f2[:, :_SLAB]
    fp = f2[:, _SLAB:]
    frb = (fr * _SC2).astype(jnp.bfloat16)

    cw = _N // _NCHUNK
    e_sum = jnp.zeros((_R, 1), dtype=jnp.float32)
    for q in range(_NCHUNK):
        fbq = fmat_ref[:, q * cw:(q + 1) * cw]
        lg = jax.lax.dot_general(frb, fbq, (((0,), (0,)), ((), ())),
                                 preferred_element_type=jnp.float32)
        e = jnp.exp2(lg.astype(jnp.bfloat16))
        e_sum = e_sum + jnp.sum(e, axis=1, keepdims=True, dtype=jnp.float32)

    a = frb.astype(jnp.float32)
    bt2 = fr.astype(jnp.bfloat16).astype(jnp.float32)
    ones_c = jnp.ones((_C, 1), dtype=jnp.float32)
    dg = jax.lax.dot_general(a * bt2, ones_c, (((0,), (0,)), ((), ())),
                             preferred_element_type=jnp.float32)
    dgb = dg.astype(jnp.bfloat16).astype(jnp.float32)
    s_off = e_sum - jnp.exp2(dgb)

    partial = jnp.sum(jnp.log(s_off)) - _INV_T * jnp.sum(fr * fp)

    @pl.when(s == 0)
    def _():
        o_ref[0, 0, 0] = partial

    @pl.when(s > 0)
    def _():
        o_ref[0, 0, 0] = o_ref[0, 0, 0] + partial


def kernel(p1, p2, w1, b1, w2, b2):
    pooled = pl.pallas_call(
        _pool_body,
        grid=(_B, _DHW // _DBLK),
        in_specs=[
            pl.BlockSpec((1, _C, _DBLK, _DHW, _DHW), lambda b, d: (b, 0, d, 0, 0)),
            pl.BlockSpec((1, _C, _DBLK, _DHW, _DHW), lambda b, d: (b, 0, d, 0, 0)),
        ],
        out_specs=pl.BlockSpec((2, 1, _C, _ND, _S, _S),
                               lambda b, d: (0, b, 0, d, 0, 0)),
        out_shape=jax.ShapeDtypeStruct((2, _B, _C, _S, _S, _S), jnp.float32),
        compiler_params=pltpu.CompilerParams(
            dimension_semantics=("arbitrary", "arbitrary"),
            vmem_limit_bytes=56 * 1024 * 1024,
        ),
        name="region_pool",
    )(p1, p2)

    pooled = pooled.reshape(2, _B, _C, _SLAB)

    partials = pl.pallas_call(
        _loss_body,
        grid=(2, _NB),
        in_specs=[
            pl.BlockSpec((2, _B, _C, _SLAB), lambda i, j: (0, 0, 0, 0)),
            pl.BlockSpec((_C, _C), lambda i, j: (0, 0)),
            pl.BlockSpec((_C, 1), lambda i, j: (0, 0)),
            pl.BlockSpec((_C, _C), lambda i, j: (0, 0)),
            pl.BlockSpec((_C, 1), lambda i, j: (0, 0)),
        ],
        out_specs=pl.BlockSpec((1, 1, 1), lambda i, j: (0, 0, 0),
                               memory_space=pltpu.SMEM),
        out_shape=jax.ShapeDtypeStruct((1, 1, 1), jnp.float32),
        scratch_shapes=[pltpu.VMEM((_C, _N), jnp.bfloat16)],
        compiler_params=pltpu.CompilerParams(
            dimension_semantics=("arbitrary", "arbitrary"),
            vmem_limit_bytes=48 * 1024 * 1024,
        ),
        name="head_infonce_loss",
    )(pooled, w1, b1.reshape(_C, 1), w2, b2.reshape(_C, 1))

    return partials[0, 0, 0] / _N


# speedup vs baseline: 1.2499x; 1.0218x over previous
import jax
import jax.numpy as jnp
from jax.experimental import pallas as pl
from jax.experimental.pallas import tpu as pltpu

_B, _C, _DHW, _S = 8, 16, 64, 8
_POOL = _DHW // _S
_SLAB = _S ** 3
_HALF = _B * _SLAB
_N = 2 * _HALF
_INV_T = 10.0
_SC2 = 14.426950408889634
_EPS = 1e-12

_DBLK = 16
_ND = _DBLK // _POOL
_R = 2 * _SLAB
_RS = _R // _SLAB
_NB = _HALF // _R
_NCHUNK = 8


def _pool_body(x1_ref, x2_ref, o_ref):
    wi = jax.lax.broadcasted_iota(jnp.int32, (_DHW, _S), 0)
    wo = jax.lax.broadcasted_iota(jnp.int32, (_DHW, _S), 1)
    pm = jnp.where(wi // _POOL == wo, 1.0 / (_POOL ** 3), 0.0).astype(jnp.float32)

    for v, x_ref in ((0, x1_ref), (1, x2_ref)):
        hs = []
        for k in range(_ND):
            acc = x_ref[0, :, _POOL * k]
            for d in range(1, _POOL):
                acc = acc + x_ref[0, :, _POOL * k + d]
            hs.append(acc.reshape(_C, _S, _POOL, _DHW).sum(axis=2))
        xh = jnp.stack(hs, axis=1)
        y = jnp.dot(xh.reshape(_C * _ND * _S, _DHW), pm,
                    preferred_element_type=jnp.float32,
                    precision=jax.lax.Precision.HIGHEST)
        o_ref[v, 0] = y.reshape(_C, _ND, _S, _S)


def _loss_body(pf_ref, w1_ref, b1_ref, w2_ref, b2_ref, o_ref, fmat_ref):
    i = pl.program_id(0)
    j = pl.program_id(1)
    s = i * _NB + j

    def _mlp(x):
        h = jnp.dot(w1_ref[...], x,
                    preferred_element_type=jnp.float32) + b1_ref[...]
        h = jnp.maximum(h, 0.0)
        f = jnp.dot(w2_ref[...], h,
                    preferred_element_type=jnp.float32) + b2_ref[...]
        nrm = jnp.sqrt(jnp.sum(f * f, axis=0, keepdims=True))
        return f / jnp.maximum(nrm, _EPS)

    @pl.when(s == 0)
    def _():
        xcat = jnp.concatenate(
            [pf_ref[v, b] for v in range(2) for b in range(_B)], axis=1)
        fmat_ref[...] = _mlp(xcat).astype(jnp.bfloat16)

    xp = jnp.concatenate(
        [pf_ref[i, _RS * j + k] for k in range(_RS)]
        + [pf_ref[1 - i, _RS * j + k] for k in range(_RS)], axis=1)
    f2 = _mlp(xp)
    fr = f2[:, :_R]
    fp = f2[:, _R:]
    frb = (fr * _SC2).astype(jnp.bfloat16)

    cw = _N // _NCHUNK
    e_sum = jnp.zeros((_R, 1), dtype=jnp.float32)
    for q in range(_NCHUNK):
        fbq = fmat_ref[:, q * cw:(q + 1) * cw]
        lg = jax.lax.dot_general(frb, fbq, (((0,), (0,)), ((), ())),
                                 preferred_element_type=jnp.float32)
        e = jnp.exp2(lg.astype(jnp.bfloat16))
        e_sum = e_sum + jnp.sum(e, axis=1, keepdims=True, dtype=jnp.float32)

    a = frb.astype(jnp.float32)
    bt2 = fr.astype(jnp.bfloat16).astype(jnp.float32)
    ones_c = jnp.ones((_C, 1), dtype=jnp.float32)
    dg = jax.lax.dot_general(a * bt2, ones_c, (((0,), (0,)), ((), ())),
                             preferred_element_type=jnp.float32)
    dgb = dg.astype(jnp.bfloat16).astype(jnp.float32)
    s_off = e_sum - jnp.exp2(dgb)

    partial = jnp.sum(jnp.log(s_off)) - _INV_T * jnp.sum(fr * fp)

    @pl.when(s == 0)
    def _():
        o_ref[0, 0, 0] = partial

    @pl.when(s > 0)
    def _():
        o_ref[0, 0, 0] = o_ref[0, 0, 0] + partial


def kernel(p1, p2, w1, b1, w2, b2):
    pooled = pl.pallas_call(
        _pool_body,
        grid=(_B, _DHW // _DBLK),
        in_specs=[
            pl.BlockSpec((1, _C, _DBLK, _DHW, _DHW), lambda b, d: (b, 0, d, 0, 0)),
            pl.BlockSpec((1, _C, _DBLK, _DHW, _DHW), lambda b, d: (b, 0, d, 0, 0)),
        ],
        out_specs=pl.BlockSpec((2, 1, _C, _ND, _S, _S),
                               lambda b, d: (0, b, 0, d, 0, 0)),
        out_shape=jax.ShapeDtypeStruct((2, _B, _C, _S, _S, _S), jnp.float32),
        compiler_params=pltpu.CompilerParams(
            dimension_semantics=("arbitrary", "arbitrary"),
            vmem_limit_bytes=56 * 1024 * 1024,
        ),
        name="region_pool",
    )(p1, p2)

    pooled = pooled.reshape(2, _B, _C, _SLAB)

    partials = pl.pallas_call(
        _loss_body,
        grid=(2, _NB),
        in_specs=[
            pl.BlockSpec((2, _B, _C, _SLAB), lambda i, j: (0, 0, 0, 0)),
            pl.BlockSpec((_C, _C), lambda i, j: (0, 0)),
            pl.BlockSpec((_C, 1), lambda i, j: (0, 0)),
            pl.BlockSpec((_C, _C), lambda i, j: (0, 0)),
            pl.BlockSpec((_C, 1), lambda i, j: (0, 0)),
        ],
        out_specs=pl.BlockSpec((1, 1, 1), lambda i, j: (0, 0, 0),
                               memory_space=pltpu.SMEM),
        out_shape=jax.ShapeDtypeStruct((1, 1, 1), jnp.float32),
        scratch_shapes=[pltpu.VMEM((_C, _N), jnp.bfloat16)],
        compiler_params=pltpu.CompilerParams(
            dimension_semantics=("arbitrary", "arbitrary"),
            vmem_limit_bytes=48 * 1024 * 1024,
        ),
        name="head_infonce_loss",
    )(pooled, w1, b1.reshape(_C, 1), w2, b2.reshape(_C, 1))

    return partials[0, 0, 0] / _N


# speedup vs baseline: 1.2648x; 1.0119x over previous
import jax
import jax.numpy as jnp
from jax.experimental import pallas as pl
from jax.experimental.pallas import tpu as pltpu

_B, _C, _DHW, _S = 8, 16, 64, 8
_POOL = _DHW // _S
_SLAB = _S ** 3
_HALF = _B * _SLAB
_N = 2 * _HALF
_INV_T = 10.0
_SC2 = 14.426950408889634
_EPS = 1e-12

_DBLK = 16
_ND = _DBLK // _POOL
_R = 4 * _SLAB
_RS = _R // _SLAB
_NB = _HALF // _R
_NCHUNK = 8


def _pool_body(x1_ref, x2_ref, o_ref):
    wi = jax.lax.broadcasted_iota(jnp.int32, (_DHW, _S), 0)
    wo = jax.lax.broadcasted_iota(jnp.int32, (_DHW, _S), 1)
    pm = jnp.where(wi // _POOL == wo, 1.0 / (_POOL ** 3), 0.0).astype(jnp.float32)

    for v, x_ref in ((0, x1_ref), (1, x2_ref)):
        hs = []
        for k in range(_ND):
            acc = x_ref[0, :, _POOL * k]
            for d in range(1, _POOL):
                acc = acc + x_ref[0, :, _POOL * k + d]
            hs.append(acc.reshape(_C, _S, _POOL, _DHW).sum(axis=2))
        xh = jnp.stack(hs, axis=1)
        y = jnp.dot(xh.reshape(_C * _ND * _S, _DHW), pm,
                    preferred_element_type=jnp.float32,
                    precision=jax.lax.Precision.HIGHEST)
        o_ref[v, 0] = y.reshape(_C, _ND, _S, _S)


def _loss_body(pf_ref, w1_ref, b1_ref, w2_ref, b2_ref, o_ref, fmat_ref):
    i = pl.program_id(0)
    j = pl.program_id(1)
    s = i * _NB + j

    def _mlp(x):
        h = jnp.dot(w1_ref[...], x,
                    preferred_element_type=jnp.float32) + b1_ref[...]
        h = jnp.maximum(h, 0.0)
        f = jnp.dot(w2_ref[...], h,
                    preferred_element_type=jnp.float32) + b2_ref[...]
        nrm = jnp.sqrt(jnp.sum(f * f, axis=0, keepdims=True))
        return f / jnp.maximum(nrm, _EPS)

    @pl.when(s == 0)
    def _():
        xcat = jnp.concatenate(
            [pf_ref[v, b] for v in range(2) for b in range(_B)], axis=1)
        fmat_ref[...] = _mlp(xcat).astype(jnp.bfloat16)

    xp = jnp.concatenate(
        [pf_ref[i, _RS * j + k] for k in range(_RS)]
        + [pf_ref[1 - i, _RS * j + k] for k in range(_RS)], axis=1)
    f2 = _mlp(xp)
    fr = f2[:, :_R]
    fp = f2[:, _R:]
    frb = (fr * _SC2).astype(jnp.bfloat16)

    cw = _N // _NCHUNK
    e_sum = jnp.zeros((_R, 1), dtype=jnp.float32)
    for q in range(_NCHUNK):
        fbq = fmat_ref[:, q * cw:(q + 1) * cw]
        lg = jax.lax.dot_general(frb, fbq, (((0,), (0,)), ((), ())),
                                 preferred_element_type=jnp.float32)
        e = jnp.exp2(lg.astype(jnp.bfloat16))
        e_sum = e_sum + jnp.sum(e, axis=1, keepdims=True, dtype=jnp.float32)

    a = frb.astype(jnp.float32)
    bt2 = fr.astype(jnp.bfloat16).astype(jnp.float32)
    ones_c = jnp.ones((_C, 1), dtype=jnp.float32)
    dg = jax.lax.dot_general(a * bt2, ones_c, (((0,), (0,)), ((), ())),
                             preferred_element_type=jnp.float32)
    dgb = dg.astype(jnp.bfloat16).astype(jnp.float32)
    s_off = e_sum - jnp.exp2(dgb)

    partial = jnp.sum(jnp.log(s_off)) - _INV_T * jnp.sum(fr * fp)

    @pl.when(s == 0)
    def _():
        o_ref[0, 0, 0] = partial

    @pl.when(s > 0)
    def _():
        o_ref[0, 0, 0] = o_ref[0, 0, 0] + partial


def kernel(p1, p2, w1, b1, w2, b2):
    pooled = pl.pallas_call(
        _pool_body,
        grid=(_B, _DHW // _DBLK),
        in_specs=[
            pl.BlockSpec((1, _C, _DBLK, _DHW, _DHW), lambda b, d: (b, 0, d, 0, 0)),
            pl.BlockSpec((1, _C, _DBLK, _DHW, _DHW), lambda b, d: (b, 0, d, 0, 0)),
        ],
        out_specs=pl.BlockSpec((2, 1, _C, _ND, _S, _S),
                               lambda b, d: (0, b, 0, d, 0, 0)),
        out_shape=jax.ShapeDtypeStruct((2, _B, _C, _S, _S, _S), jnp.float32),
        compiler_params=pltpu.CompilerParams(
            dimension_semantics=("arbitrary", "arbitrary"),
            vmem_limit_bytes=56 * 1024 * 1024,
        ),
        name="region_pool",
    )(p1, p2)

    pooled = pooled.reshape(2, _B, _C, _SLAB)

    partials = pl.pallas_call(
        _loss_body,
        grid=(2, _NB),
        in_specs=[
            pl.BlockSpec((2, _B, _C, _SLAB), lambda i, j: (0, 0, 0, 0)),
            pl.BlockSpec((_C, _C), lambda i, j: (0, 0)),
            pl.BlockSpec((_C, 1), lambda i, j: (0, 0)),
            pl.BlockSpec((_C, _C), lambda i, j: (0, 0)),
            pl.BlockSpec((_C, 1), lambda i, j: (0, 0)),
        ],
        out_specs=pl.BlockSpec((1, 1, 1), lambda i, j: (0, 0, 0),
                               memory_space=pltpu.SMEM),
        out_shape=jax.ShapeDtypeStruct((1, 1, 1), jnp.float32),
        scratch_shapes=[pltpu.VMEM((_C, _N), jnp.bfloat16)],
        compiler_params=pltpu.CompilerParams(
            dimension_semantics=("arbitrary", "arbitrary"),
            vmem_limit_bytes=48 * 1024 * 1024,
        ),
        name="head_infonce_loss",
    )(pooled, w1, b1.reshape(_C, 1), w2, b2.reshape(_C, 1))

    return partials[0, 0, 0] / _N
